# baseline (speedup 1.0000x reference)
import jax
import jax.numpy as jnp
from jax import lax
from jax.experimental import pallas as pl

T, XD, KD, E = 2048, 1024, 512, 8
TT = 512


def _body(gate_ref, raw_ref, keys_ref, wg_ref, we_ref, out_ref, scores_ref):
    j = pl.program_id(1)

    @pl.when(j == 0)
    def _gate():
        q = lax.dot_general(
            gate_ref[...], wg_ref[...], (((1,), (0,)), ((), ())),
            preferred_element_type=jnp.float32)
        logits = lax.dot_general(
            q, keys_ref[...], (((1,), (1,)), ((), ())),
            preferred_element_type=jnp.float32)
        lane = lax.broadcasted_iota(jnp.int32, (TT, E), 1)
        m1 = jnp.max(logits, axis=1, keepdims=True)
        idx1 = jnp.min(jnp.where(logits == m1, lane, E), axis=1, keepdims=True)
        rest = jnp.where(lane == idx1, -jnp.inf, logits)
        m2 = jnp.max(rest, axis=1, keepdims=True)
        idx2 = jnp.min(jnp.where(rest == m2, lane, E), axis=1, keepdims=True)
        ex = jnp.exp(m2 - m1)
        g1 = 1.0 / (1.0 + ex)
        g2 = ex * g1
        scores_ref[...] = (jnp.where(lane == idx1, g1, 0.0)
                           + jnp.where(lane == idx2, g2, 0.0))

    lane = lax.broadcasted_iota(jnp.int32, (TT, E), 1)
    sc = scores_ref[...]
    contrib = None
    for u in range(4):
        cu = jnp.sum(jnp.where(lane == 4 * j + u, sc, 0.0),
                     axis=1, keepdims=True)
        term = cu * lax.dot_general(
            raw_ref[...], we_ref[u], (((1,), (0,)), ((), ())),
            preferred_element_type=jnp.float32)
        contrib = term if contrib is None else contrib + term

    @pl.when(j == 0)
    def _init():
        out_ref[...] = contrib

    @pl.when(j > 0)
    def _acc():
        out_ref[...] += contrib


def kernel(gate_inputs, raw_inputs, keys, W_gate, W_expert):
    out, scores = pl.pallas_call(
        _body,
        grid=(T // TT, E // 4),
        in_specs=[
            pl.BlockSpec((TT, XD), lambda i, j: (i, 0)),
            pl.BlockSpec((TT, XD), lambda i, j: (i, 0)),
            pl.BlockSpec((E, KD), lambda i, j: (0, 0)),
            pl.BlockSpec((XD, KD), lambda i, j: (0, 0)),
            pl.BlockSpec((4, XD, XD), lambda i, j: (j, 0, 0)),
        ],
        out_specs=[
            pl.BlockSpec((TT, XD), lambda i, j: (i, 0)),
            pl.BlockSpec((TT, E), lambda i, j: (i, 0)),
        ],
        out_shape=[
            jax.ShapeDtypeStruct((T, XD), jnp.float32),
            jax.ShapeDtypeStruct((T, E), jnp.float32),
        ],
    )(gate_inputs, raw_inputs, keys, W_gate, W_expert)
    return out, scores


# speedup vs baseline: 1.1686x; 1.1686x over previous
import jax
import jax.numpy as jnp
from jax import lax
from jax.experimental import pallas as pl

T, XD, KD, E = 2048, 1024, 512, 8
TT = 1024


def _body(gate_ref, raw_ref, keys_ref, wg_ref, we_ref, out_ref, scores_ref):
    j = pl.program_id(1)

    @pl.when(j == 0)
    def _gate():
        q = lax.dot_general(
            gate_ref[...], wg_ref[...], (((1,), (0,)), ((), ())),
            preferred_element_type=jnp.float32)
        logits = lax.dot_general(
            q, keys_ref[...], (((1,), (1,)), ((), ())),
            preferred_element_type=jnp.float32)
        lane = lax.broadcasted_iota(jnp.int32, (TT, E), 1)
        m1 = jnp.max(logits, axis=1, keepdims=True)
        idx1 = jnp.min(jnp.where(logits == m1, lane, E), axis=1, keepdims=True)
        rest = jnp.where(lane == idx1, -jnp.inf, logits)
        m2 = jnp.max(rest, axis=1, keepdims=True)
        idx2 = jnp.min(jnp.where(rest == m2, lane, E), axis=1, keepdims=True)
        ex = jnp.exp(m2 - m1)
        g1 = 1.0 / (1.0 + ex)
        g2 = ex * g1
        scores_ref[...] = (jnp.where(lane == idx1, g1, 0.0)
                           + jnp.where(lane == idx2, g2, 0.0))

    lane = lax.broadcasted_iota(jnp.int32, (TT, E), 1)
    sc = scores_ref[...]
    contrib = None
    for u in range(2):
        cu = jnp.sum(jnp.where(lane == 2 * j + u, sc, 0.0),
                     axis=1, keepdims=True)
        term = cu * lax.dot_general(
            raw_ref[...], we_ref[u], (((1,), (0,)), ((), ())),
            preferred_element_type=jnp.float32)
        contrib = term if contrib is None else contrib + term

    @pl.when(j == 0)
    def _init():
        out_ref[...] = contrib

    @pl.when(j > 0)
    def _acc():
        out_ref[...] += contrib


def kernel(gate_inputs, raw_inputs, keys, W_gate, W_expert):
    out, scores = pl.pallas_call(
        _body,
        grid=(T // TT, E // 2),
        in_specs=[
            pl.BlockSpec((TT, XD), lambda i, j: (i, 0)),
            pl.BlockSpec((TT, XD), lambda i, j: (i, 0)),
            pl.BlockSpec((E, KD), lambda i, j: (0, 0)),
            pl.BlockSpec((XD, KD), lambda i, j: (0, 0)),
            pl.BlockSpec((2, XD, XD), lambda i, j: (j, 0, 0)),
        ],
        out_specs=[
            pl.BlockSpec((TT, XD), lambda i, j: (i, 0)),
            pl.BlockSpec((TT, E), lambda i, j: (i, 0)),
        ],
        out_shape=[
            jax.ShapeDtypeStruct((T, XD), jnp.float32),
            jax.ShapeDtypeStruct((T, E), jnp.float32),
        ],
    )(gate_inputs, raw_inputs, keys, W_gate, W_expert)
    return out, scores
